# baseline (speedup 1.0000x reference)
import functools

import jax
import jax.numpy as jnp
import numpy as np
from jax import lax
from jax.experimental import pallas as pl
from jax.experimental.pallas import tpu as pltpu
from jax.experimental.pallas import tpu_sc as plsc

TOKENS = 98

NW = 32
CHUNK = 72


def _uniform_threefry_np(seed, rows, cols):
    size = rows * cols
    i = np.arange(size, dtype=np.uint64)
    x = [
        (i >> np.uint64(32)).astype(np.uint32),
        (i & np.uint64(0xFFFFFFFF)).astype(np.uint32),
    ]
    k0 = np.uint32(seed >> 32)
    k1 = np.uint32(seed & 0xFFFFFFFF)
    ks = [k0, k1, np.uint32(k0 ^ k1 ^ np.uint32(0x1BD11BDA))]
    rotations = [
        np.array([13, 15, 26, 6], dtype=np.uint32),
        np.array([17, 29, 16, 24], dtype=np.uint32),
    ]

    def rotl(v, d):
        return (v << d) | (v >> np.uint32(32 - int(d)))

    x[0] = x[0] + ks[0]
    x[1] = x[1] + ks[1]
    for r5 in range(5):
        for r in rotations[r5 % 2]:
            x[0] = x[0] + x[1]
            x[1] = x[0] ^ rotl(x[1], r)
        x[0] = x[0] + ks[(r5 + 1) % 3]
        x[1] = x[1] + ks[(r5 + 2) % 3] + np.uint32(r5 + 1)
    bits = (x[0] ^ x[1]).reshape(rows, cols)
    float_bits = (bits >> np.uint32(9)) | np.uint32(0x3F800000)
    floats = float_bits.view(np.float32) - np.float32(1.0)
    return np.maximum(np.float32(0.0), floats)


def _selected_token_indices(N, L):
    noise = _uniform_threefry_np(1, N, L - 1)
    patch_mask = np.argsort(noise, axis=1, kind="stable") + 1
    patch_mask = np.sort(patch_mask[:, :TOKENS], axis=1)
    cls_mask = np.zeros((N, 1), dtype=patch_mask.dtype)
    return np.concatenate([cls_mask, patch_mask], axis=1)


def _make_gather(B, D, nchunks):
    mesh = plsc.VectorSubcoreMesh(core_axis_name="c", subcore_axis_name="s")
    b_per_w = B // NW

    NBUF = 4
    HALF = D // 2
    nunits = nchunks * 2

    @functools.partial(
        pl.kernel,
        mesh=mesh,
        out_type=jax.ShapeDtypeStruct((B, D), jnp.float32),
        scratch_types=[
            pltpu.VMEM((nchunks, CHUNK), jnp.int32),
        ]
        + [pltpu.VMEM((CHUNK, HALF), jnp.float32)] * NBUF
        + [pltpu.SemaphoreType.DMA] * (2 * NBUF),
    )
    def gather_rows(table_hbm, idx_hbm, out_hbm, idx_v, *bufs_sems):
        bufs = bufs_sems[:NBUF]
        gsems = bufs_sems[NBUF:2 * NBUF]
        ssems = bufs_sems[2 * NBUF:]
        wid = lax.axis_index("s") * 2 + lax.axis_index("c")
        base = wid * b_per_w
        pltpu.sync_copy(idx_hbm.at[wid], idx_v)

        def gsrc(u):
            j, h = u // 2, u % 2
            return table_hbm.at[idx_v.at[j], pl.ds(h * HALF, HALF)]

        def sdst(u):
            j, h = u // 2, u % 2
            return out_hbm.at[pl.ds(base + j * CHUNK, CHUNK),
                              pl.ds(h * HALF, HALF)]

        gh = [None] * NBUF
        sh = [None] * NBUF
        gh[0] = pltpu.async_copy(gsrc(0), bufs[0], gsems[0])
        gh[1] = pltpu.async_copy(gsrc(1), bufs[1], gsems[1])
        for u in range(nunits):
            b = u % NBUF
            bn = (u + 2) % NBUF
            if u + 2 < nunits:
                if sh[bn] is not None:
                    sh[bn].wait()
                gh[bn] = pltpu.async_copy(gsrc(u + 2), bufs[bn], gsems[bn])
            gh[b].wait()
            sh[b] = pltpu.async_copy(bufs[b], sdst(u), ssems[b])
        for b in range(NBUF):
            if sh[b] is not None:
                sh[b].wait()

    return gather_rows


def kernel(x):
    N, L, D = x.shape
    T = TOKENS + 1
    mask = _selected_token_indices(N, L)
    B = N * T
    src = mask.T.astype(np.int64) * N + np.arange(N, dtype=np.int64)[None, :]
    nchunks = B // (NW * CHUNK)
    idx3 = jnp.asarray(src.reshape(NW, nchunks, CHUNK).astype(np.int32))
    xt = jnp.transpose(x, (1, 0, 2)).reshape(L * N, D)
    out2 = _make_gather(B, D, nchunks)(xt, idx3)
    return jnp.transpose(out2.reshape(T, N, D), (1, 0, 2))


# speedup vs baseline: 1.0227x; 1.0227x over previous
import functools

import jax
import jax.numpy as jnp
import numpy as np
from jax import lax
from jax.experimental import pallas as pl
from jax.experimental.pallas import tpu as pltpu
from jax.experimental.pallas import tpu_sc as plsc

TOKENS = 98

NW = 32
CHUNK = 72


def _uniform_threefry_np(seed, rows, cols):
    size = rows * cols
    i = np.arange(size, dtype=np.uint64)
    x = [
        (i >> np.uint64(32)).astype(np.uint32),
        (i & np.uint64(0xFFFFFFFF)).astype(np.uint32),
    ]
    k0 = np.uint32(seed >> 32)
    k1 = np.uint32(seed & 0xFFFFFFFF)
    ks = [k0, k1, np.uint32(k0 ^ k1 ^ np.uint32(0x1BD11BDA))]
    rotations = [
        np.array([13, 15, 26, 6], dtype=np.uint32),
        np.array([17, 29, 16, 24], dtype=np.uint32),
    ]

    def rotl(v, d):
        return (v << d) | (v >> np.uint32(32 - int(d)))

    x[0] = x[0] + ks[0]
    x[1] = x[1] + ks[1]
    for r5 in range(5):
        for r in rotations[r5 % 2]:
            x[0] = x[0] + x[1]
            x[1] = x[0] ^ rotl(x[1], r)
        x[0] = x[0] + ks[(r5 + 1) % 3]
        x[1] = x[1] + ks[(r5 + 2) % 3] + np.uint32(r5 + 1)
    bits = (x[0] ^ x[1]).reshape(rows, cols)
    float_bits = (bits >> np.uint32(9)) | np.uint32(0x3F800000)
    floats = float_bits.view(np.float32) - np.float32(1.0)
    return np.maximum(np.float32(0.0), floats)


def _selected_token_indices(N, L):
    noise = _uniform_threefry_np(1, N, L - 1)
    patch_mask = np.argsort(noise, axis=1, kind="stable") + 1
    patch_mask = np.sort(patch_mask[:, :TOKENS], axis=1)
    cls_mask = np.zeros((N, 1), dtype=patch_mask.dtype)
    return np.concatenate([cls_mask, patch_mask], axis=1)


def _make_gather(B, D, nchunks):
    mesh = plsc.VectorSubcoreMesh(core_axis_name="c", subcore_axis_name="s")
    b_per_w = B // NW

    @functools.partial(
        pl.kernel,
        mesh=mesh,
        out_type=jax.ShapeDtypeStruct((B, D), jnp.float32),
        scratch_types=[
            pltpu.VMEM((nchunks, CHUNK), jnp.int32),
            pltpu.VMEM((CHUNK, D), jnp.float32),
            pltpu.VMEM((CHUNK, D), jnp.float32),
            pltpu.SemaphoreType.DMA,
            pltpu.SemaphoreType.DMA,
            pltpu.SemaphoreType.DMA,
            pltpu.SemaphoreType.DMA,
        ],
    )
    def gather_rows(table_hbm, idx_hbm, out_hbm, idx_v,
                    buf0, buf1, g0, g1, s0, s1):
        wid = lax.axis_index("s") * 2 + lax.axis_index("c")
        base = wid * b_per_w
        bufs, gsems, ssems = (buf0, buf1), (g0, g1), (s0, s1)
        pltpu.sync_copy(idx_hbm.at[wid], idx_v)
        gh = [None, None]
        sh = [None, None]
        gh[0] = pltpu.async_copy(table_hbm.at[idx_v.at[0]], bufs[0], gsems[0])
        for j in range(nchunks):
            p, q = j % 2, (j + 1) % 2
            if j + 1 < nchunks:
                if sh[q] is not None:
                    sh[q].wait()
                gh[q] = pltpu.async_copy(
                    table_hbm.at[idx_v.at[j + 1]], bufs[q], gsems[q])
            gh[p].wait()
            sh[p] = pltpu.async_copy(
                bufs[p], out_hbm.at[pl.ds(base + j * CHUNK, CHUNK)], ssems[p])
        sh[(nchunks - 1) % 2].wait()
        if nchunks > 1:
            sh[(nchunks - 2) % 2].wait()

    return gather_rows


def kernel(x):
    N, L, D = x.shape
    T = TOKENS + 1
    mask = _selected_token_indices(N, L)
    B = N * T
    src = mask.T.astype(np.int64) * N + np.arange(N, dtype=np.int64)[None, :]
    nchunks = B // (NW * CHUNK)
    idx3 = jnp.asarray(src.reshape(NW, nchunks, CHUNK).astype(np.int32))
    xt = jnp.transpose(x, (1, 0, 2)).reshape(L * N, D)
    out2 = _make_gather(B, D, nchunks)(xt, idx3)
    return jnp.transpose(out2.reshape(T, N, D), (1, 0, 2))
